# baseline (speedup 1.0000x reference)
import jax
import jax.numpy as jnp
from jax.experimental import pallas as pl
from jax.experimental.pallas import tpu as pltpu

_BLOCK = 25000


def _gcn_kernel(x_ref, adj_ref, w_ref, y_ref):
    ct = jax.lax.dot_general(
        w_ref[...],
        adj_ref[...],
        (((1,), (1,)), ((), ())),
        preferred_element_type=jnp.float32,
        precision=jax.lax.Precision.HIGHEST,
    )
    y_ref[...] = jax.lax.dot_general(
        x_ref[...],
        ct,
        (((1,), (0,)), ((), ())),
        preferred_element_type=jnp.float32,
    )


def kernel(input, adj, weight):
    x = input.reshape(-1, weight.shape[0])
    n = x.shape[0]
    m = adj.shape[0]
    y = pl.pallas_call(
        _gcn_kernel,
        grid=(pl.cdiv(n, _BLOCK),),
        in_specs=[
            pl.BlockSpec((_BLOCK, x.shape[1]), lambda i: (i, 0)),
            pl.BlockSpec(adj.shape, lambda i: (0, 0)),
            pl.BlockSpec(weight.shape, lambda i: (0, 0)),
        ],
        out_specs=pl.BlockSpec((_BLOCK, m), lambda i: (i, 0)),
        out_shape=jax.ShapeDtypeStruct((n, m), jnp.float32),
        compiler_params=pltpu.CompilerParams(
            dimension_semantics=("parallel",),
        ),
    )(x, adj, weight)
    return (y.T, weight)


# speedup vs baseline: 1.0706x; 1.0706x over previous
import jax
import jax.numpy as jnp
from jax.experimental import pallas as pl
from jax.experimental.pallas import tpu as pltpu

_BLOCK = 20000


def _gcn_kernel(x_ref, adj_ref, w_ref, y_ref):
    ct = jax.lax.dot_general(
        w_ref[...],
        adj_ref[...],
        (((1,), (1,)), ((), ())),
        preferred_element_type=jnp.float32,
        precision=jax.lax.Precision.HIGHEST,
    )
    y_ref[...] = jax.lax.dot_general(
        x_ref[...],
        ct,
        (((1,), (0,)), ((), ())),
        preferred_element_type=jnp.float32,
    )


def kernel(input, adj, weight):
    x = input.reshape(-1, weight.shape[0])
    n = x.shape[0]
    m = adj.shape[0]
    y = pl.pallas_call(
        _gcn_kernel,
        grid=(pl.cdiv(n, _BLOCK),),
        in_specs=[
            pl.BlockSpec((_BLOCK, x.shape[1]), lambda i: (i, 0)),
            pl.BlockSpec(adj.shape, lambda i: (0, 0)),
            pl.BlockSpec(weight.shape, lambda i: (0, 0)),
        ],
        out_specs=pl.BlockSpec((_BLOCK, m), lambda i: (i, 0)),
        out_shape=jax.ShapeDtypeStruct((n, m), jnp.float32),
        compiler_params=pltpu.CompilerParams(
            dimension_semantics=("parallel",),
        ),
    )(x, adj, weight)
    return (y.T, weight)


# speedup vs baseline: 1.0866x; 1.0149x over previous
import jax
import jax.numpy as jnp
from jax.experimental import pallas as pl
from jax.experimental.pallas import tpu as pltpu

_BLOCK = 20000


def _gcn_kernel(x_ref, adj_ref, w_ref, y_ref, w_out_ref):
    w_out_ref[...] = w_ref[...]
    ct = jax.lax.dot_general(
        w_ref[...],
        adj_ref[...],
        (((1,), (1,)), ((), ())),
        preferred_element_type=jnp.float32,
        precision=jax.lax.Precision.HIGHEST,
    )
    y_ref[...] = jax.lax.dot_general(
        x_ref[...],
        ct,
        (((1,), (0,)), ((), ())),
        preferred_element_type=jnp.float32,
    )


def kernel(input, adj, weight):
    x = input.reshape(-1, weight.shape[0])
    n = x.shape[0]
    m = adj.shape[0]
    y, w_copy = pl.pallas_call(
        _gcn_kernel,
        grid=(pl.cdiv(n, _BLOCK),),
        in_specs=[
            pl.BlockSpec((_BLOCK, x.shape[1]), lambda i: (i, 0)),
            pl.BlockSpec(adj.shape, lambda i: (0, 0)),
            pl.BlockSpec(weight.shape, lambda i: (0, 0)),
        ],
        out_specs=[
            pl.BlockSpec((_BLOCK, m), lambda i: (i, 0)),
            pl.BlockSpec(weight.shape, lambda i: (0, 0)),
        ],
        out_shape=[
            jax.ShapeDtypeStruct((n, m), jnp.float32),
            jax.ShapeDtypeStruct(weight.shape, weight.dtype),
        ],
        compiler_params=pltpu.CompilerParams(
            dimension_semantics=("parallel",),
        ),
    )(x, adj, weight)
    return (y.T, w_copy)


# speedup vs baseline: 1.1078x; 1.0195x over previous
import jax
import jax.numpy as jnp
from jax.experimental import pallas as pl
from jax.experimental.pallas import tpu as pltpu

_BLOCK = 20000


def _gcn_kernel(x_ref, adj_ref, w_ref, y_ref, w_out_ref, ct_ref):
    @pl.when(pl.program_id(0) == 0)
    def _():
        w_out_ref[...] = w_ref[...]
        ct_ref[...] = jax.lax.dot_general(
            w_ref[...],
            adj_ref[...],
            (((1,), (1,)), ((), ())),
            preferred_element_type=jnp.float32,
            precision=jax.lax.Precision.HIGHEST,
        )

    y_ref[...] = jax.lax.dot_general(
        x_ref[...],
        ct_ref[...],
        (((1,), (0,)), ((), ())),
        preferred_element_type=jnp.float32,
    )


def kernel(input, adj, weight):
    x = input.reshape(-1, weight.shape[0])
    n = x.shape[0]
    m = adj.shape[0]
    y, w_copy = pl.pallas_call(
        _gcn_kernel,
        grid=(pl.cdiv(n, _BLOCK),),
        in_specs=[
            pl.BlockSpec((_BLOCK, x.shape[1]), lambda i: (i, 0)),
            pl.BlockSpec(adj.shape, lambda i: (0, 0)),
            pl.BlockSpec(weight.shape, lambda i: (0, 0)),
        ],
        out_specs=[
            pl.BlockSpec((_BLOCK, m), lambda i: (i, 0)),
            pl.BlockSpec(weight.shape, lambda i: (0, 0)),
        ],
        out_shape=[
            jax.ShapeDtypeStruct((n, m), jnp.float32),
            jax.ShapeDtypeStruct(weight.shape, weight.dtype),
        ],
        scratch_shapes=[pltpu.VMEM((x.shape[1], m), jnp.float32)],
        compiler_params=pltpu.CompilerParams(
            dimension_semantics=("arbitrary",),
        ),
    )(x, adj, weight)
    return (y.T, w_copy)


# speedup vs baseline: 1.1113x; 1.0032x over previous
import jax
import jax.numpy as jnp
from jax.experimental import pallas as pl
from jax.experimental.pallas import tpu as pltpu

_BLOCK = 20000


def _gcn_kernel(x_ref, adj_ref, w_ref, y_ref, w_out_ref, ct_ref):
    @pl.when(pl.program_id(0) == 0)
    def _():
        w_out_ref[...] = w_ref[...]
        ct_ref[...] = jax.lax.dot_general(
            w_ref[...],
            adj_ref[...],
            (((1,), (1,)), ((), ())),
            preferred_element_type=jnp.float32,
        )

    y_ref[...] = jax.lax.dot_general(
        x_ref[...],
        ct_ref[...],
        (((1,), (0,)), ((), ())),
        preferred_element_type=jnp.float32,
    )


def kernel(input, adj, weight):
    x = input.reshape(-1, weight.shape[0])
    n = x.shape[0]
    m = adj.shape[0]
    y, w_copy = pl.pallas_call(
        _gcn_kernel,
        grid=(pl.cdiv(n, _BLOCK),),
        in_specs=[
            pl.BlockSpec((_BLOCK, x.shape[1]), lambda i: (i, 0)),
            pl.BlockSpec(adj.shape, lambda i: (0, 0)),
            pl.BlockSpec(weight.shape, lambda i: (0, 0)),
        ],
        out_specs=[
            pl.BlockSpec((_BLOCK, m), lambda i: (i, 0)),
            pl.BlockSpec(weight.shape, lambda i: (0, 0)),
        ],
        out_shape=[
            jax.ShapeDtypeStruct((n, m), jnp.float32),
            jax.ShapeDtypeStruct(weight.shape, weight.dtype),
        ],
        scratch_shapes=[pltpu.VMEM((x.shape[1], m), jnp.float32)],
        compiler_params=pltpu.CompilerParams(
            dimension_semantics=("arbitrary",),
        ),
    )(x, adj, weight)
    return (y.T, w_copy)


# speedup vs baseline: 1.1439x; 1.0293x over previous
import jax
import jax.numpy as jnp
from jax.experimental import pallas as pl
from jax.experimental.pallas import tpu as pltpu

_BLOCK = 24576


def _gcn_kernel(x_ref, adj_ref, w_ref, y_ref, w_out_ref, ct_ref):
    @pl.when(pl.program_id(0) == 0)
    def _():
        w_out_ref[...] = w_ref[...]
        ct_ref[...] = jax.lax.dot_general(
            w_ref[...],
            adj_ref[...],
            (((1,), (1,)), ((), ())),
            preferred_element_type=jnp.float32,
        )

    y_ref[...] = jax.lax.dot_general(
        x_ref[...],
        ct_ref[...],
        (((1,), (0,)), ((), ())),
        preferred_element_type=jnp.float32,
    )


def kernel(input, adj, weight):
    x = input.reshape(-1, weight.shape[0])
    n = x.shape[0]
    m = adj.shape[0]
    y, w_copy = pl.pallas_call(
        _gcn_kernel,
        grid=(pl.cdiv(n, _BLOCK),),
        in_specs=[
            pl.BlockSpec((_BLOCK, x.shape[1]), lambda i: (i, 0)),
            pl.BlockSpec(adj.shape, lambda i: (0, 0)),
            pl.BlockSpec(weight.shape, lambda i: (0, 0)),
        ],
        out_specs=[
            pl.BlockSpec((_BLOCK, m), lambda i: (i, 0)),
            pl.BlockSpec(weight.shape, lambda i: (0, 0)),
        ],
        out_shape=[
            jax.ShapeDtypeStruct((n, m), jnp.float32),
            jax.ShapeDtypeStruct(weight.shape, weight.dtype),
        ],
        scratch_shapes=[pltpu.VMEM((x.shape[1], m), jnp.float32)],
        compiler_params=pltpu.CompilerParams(
            dimension_semantics=("arbitrary",),
        ),
    )(x, adj, weight)
    return (y.T, w_copy)


# speedup vs baseline: 1.1743x; 1.0266x over previous
import jax
import jax.numpy as jnp
from jax.experimental import pallas as pl
from jax.experimental.pallas import tpu as pltpu

_BLOCK = 28672


def _gcn_kernel(x_ref, adj_ref, w_ref, y_ref, w_out_ref, ct_ref):
    @pl.when(pl.program_id(0) == 0)
    def _():
        w_out_ref[...] = w_ref[...]
        ct_ref[...] = jax.lax.dot_general(
            w_ref[...],
            adj_ref[...],
            (((1,), (1,)), ((), ())),
            preferred_element_type=jnp.float32,
        )

    y_ref[...] = jax.lax.dot_general(
        x_ref[...],
        ct_ref[...],
        (((1,), (0,)), ((), ())),
        preferred_element_type=jnp.float32,
    )


def kernel(input, adj, weight):
    x = input.reshape(-1, weight.shape[0])
    n = x.shape[0]
    m = adj.shape[0]
    y, w_copy = pl.pallas_call(
        _gcn_kernel,
        grid=(pl.cdiv(n, _BLOCK),),
        in_specs=[
            pl.BlockSpec((_BLOCK, x.shape[1]), lambda i: (i, 0)),
            pl.BlockSpec(adj.shape, lambda i: (0, 0)),
            pl.BlockSpec(weight.shape, lambda i: (0, 0)),
        ],
        out_specs=[
            pl.BlockSpec((_BLOCK, m), lambda i: (i, 0)),
            pl.BlockSpec(weight.shape, lambda i: (0, 0)),
        ],
        out_shape=[
            jax.ShapeDtypeStruct((n, m), jnp.float32),
            jax.ShapeDtypeStruct(weight.shape, weight.dtype),
        ],
        scratch_shapes=[pltpu.VMEM((x.shape[1], m), jnp.float32)],
        compiler_params=pltpu.CompilerParams(
            dimension_semantics=("arbitrary",),
        ),
    )(x, adj, weight)
    return (y.T, w_copy)


# speedup vs baseline: 1.1776x; 1.0028x over previous
import jax
import jax.numpy as jnp
from jax.experimental import pallas as pl
from jax.experimental.pallas import tpu as pltpu

_BLOCK = 29696


def _gcn_kernel(x_ref, adj_ref, w_ref, y_ref, w_out_ref, ct_ref):
    @pl.when(pl.program_id(0) == 0)
    def _():
        w_out_ref[...] = w_ref[...]
        ct_ref[...] = jax.lax.dot_general(
            w_ref[...],
            adj_ref[...],
            (((1,), (1,)), ((), ())),
            preferred_element_type=jnp.float32,
        )

    y_ref[...] = jax.lax.dot_general(
        x_ref[...],
        ct_ref[...],
        (((1,), (0,)), ((), ())),
        preferred_element_type=jnp.float32,
    )


def kernel(input, adj, weight):
    x = input.reshape(-1, weight.shape[0])
    n = x.shape[0]
    m = adj.shape[0]
    y, w_copy = pl.pallas_call(
        _gcn_kernel,
        grid=(pl.cdiv(n, _BLOCK),),
        in_specs=[
            pl.BlockSpec((_BLOCK, x.shape[1]), lambda i: (i, 0)),
            pl.BlockSpec(adj.shape, lambda i: (0, 0)),
            pl.BlockSpec(weight.shape, lambda i: (0, 0)),
        ],
        out_specs=[
            pl.BlockSpec((_BLOCK, m), lambda i: (i, 0)),
            pl.BlockSpec(weight.shape, lambda i: (0, 0)),
        ],
        out_shape=[
            jax.ShapeDtypeStruct((n, m), jnp.float32),
            jax.ShapeDtypeStruct(weight.shape, weight.dtype),
        ],
        scratch_shapes=[pltpu.VMEM((x.shape[1], m), jnp.float32)],
        compiler_params=pltpu.CompilerParams(
            dimension_semantics=("arbitrary",),
        ),
    )(x, adj, weight)
    return (y.T, w_copy)
